# baseline (speedup 1.0000x reference)
import jax
import jax.numpy as jnp
from jax.experimental import pallas as pl


def _body(sym_ref, emb_ref, pe_ref, out_ref):
    mask = (sym_ref[0] != 0).astype(jnp.float32)
    out_ref[0] = emb_ref[0] + pe_ref[...] * mask


def kernel(embedded, symbol, pe):
    B, S, D = embedded.shape
    BS = 256
    n_s = S // BS
    sym3 = symbol.astype(jnp.int32).reshape(B, S, 1)
    return pl.pallas_call(
        _body,
        grid=(n_s, B),
        in_specs=[
            pl.BlockSpec((1, BS, 1), lambda s, b: (b, s, 0)),
            pl.BlockSpec((1, BS, D), lambda s, b: (b, s, 0)),
            pl.BlockSpec((BS, D), lambda s, b: (s, 0)),
        ],
        out_specs=pl.BlockSpec((1, BS, D), lambda s, b: (b, s, 0)),
        out_shape=jax.ShapeDtypeStruct((B, S, D), jnp.float32),
    )(sym3, embedded, pe)


# speedup vs baseline: 1.4281x; 1.4281x over previous
import jax
import jax.numpy as jnp
from jax.experimental import pallas as pl


def _body(sym_ref, emb_ref, pe_ref, out_ref):
    mask = (sym_ref[0] != 0).astype(jnp.float32)
    out_ref[0] = emb_ref[0] + pe_ref[...] * mask


def kernel(embedded, symbol, pe):
    B, S, D = embedded.shape
    BS = 2048
    n_s = S // BS
    sym3 = symbol.astype(jnp.int32).reshape(B, S, 1)
    return pl.pallas_call(
        _body,
        grid=(n_s, B),
        in_specs=[
            pl.BlockSpec((1, BS, 1), lambda s, b: (b, s, 0)),
            pl.BlockSpec((1, BS, D), lambda s, b: (b, s, 0)),
            pl.BlockSpec((BS, D), lambda s, b: (s, 0)),
        ],
        out_specs=pl.BlockSpec((1, BS, D), lambda s, b: (b, s, 0)),
        out_shape=jax.ShapeDtypeStruct((B, S, D), jnp.float32),
    )(sym3, embedded, pe)


# speedup vs baseline: 1.6897x; 1.1832x over previous
import jax
import jax.numpy as jnp
from jax import lax
from jax.experimental import pallas as pl


def _body(sym_ref, emb_ref, pe_ref, out_ref):
    b = pl.program_id(1)
    row = sym_ref[pl.ds(b, 1), :]
    mask = (lax.transpose(row, (1, 0)) != 0).astype(jnp.float32)
    out_ref[0] = emb_ref[0] + pe_ref[...] * mask


def kernel(embedded, symbol, pe):
    B, S, D = embedded.shape
    BS = 2048
    n_s = S // BS
    sym2 = symbol.astype(jnp.int32)
    return pl.pallas_call(
        _body,
        grid=(n_s, B),
        in_specs=[
            pl.BlockSpec((B, S), lambda s, b: (0, 0)),
            pl.BlockSpec((1, BS, D), lambda s, b: (b, s, 0)),
            pl.BlockSpec((BS, D), lambda s, b: (s, 0)),
        ],
        out_specs=pl.BlockSpec((1, BS, D), lambda s, b: (b, s, 0)),
        out_shape=jax.ShapeDtypeStruct((B, S, D), jnp.float32),
    )(sym2, embedded, pe)
